# Initial kernel scaffold; baseline (speedup 1.0000x reference)
import jax
import jax.numpy as jnp
from jax.experimental import pallas as pl


def kernel(xyzs, features, params):
    raise NotImplementedError("write your pallas kernel here")


# speedup vs baseline: 8.2966x; 8.2966x over previous
import functools

import jax
import jax.numpy as jnp
import numpy as np
from jax.experimental import pallas as pl
from jax.experimental.pallas import tpu as pltpu

RADIUS = 0.3
K = 16
B, T, N = 2, 4, 1024
M1, M2 = 128, 32
EMBED = 1024
HIGH = jax.lax.Precision.HIGHEST
EPS = 1e-5


def _fps_body(x_ref, y_ref, z_ref, ax_ref, ay_ref, az_ref, *, npoint):
    x = x_ref[...]
    y = y_ref[...]
    z = z_ref[...]
    f, n = x.shape
    col = jax.lax.broadcasted_iota(jnp.int32, (f, npoint), 1)
    lane = jax.lax.broadcasted_iota(jnp.int32, (f, n), 1)
    cx = x[:, 0:1]
    cy = y[:, 0:1]
    cz = z[:, 0:1]
    zero = jnp.zeros((f, npoint), jnp.float32)
    ax = jnp.where(col == 0, cx, zero)
    ay = jnp.where(col == 0, cy, zero)
    az = jnp.where(col == 0, cz, zero)

    def body(i, carry):
        dists, cx, cy, cz, ax, ay, az = carry
        d = (x - cx) ** 2 + (y - cy) ** 2 + (z - cz) ** 2
        dists = jnp.minimum(dists, d)
        m = jnp.max(dists, axis=1, keepdims=True)
        idx = jnp.min(jnp.where(dists == m, lane, n), axis=1, keepdims=True)
        oh = lane == idx
        cx = jnp.sum(jnp.where(oh, x, 0.0), axis=1, keepdims=True)
        cy = jnp.sum(jnp.where(oh, y, 0.0), axis=1, keepdims=True)
        cz = jnp.sum(jnp.where(oh, z, 0.0), axis=1, keepdims=True)
        ax = jnp.where(col == i, cx, ax)
        ay = jnp.where(col == i, cy, ay)
        az = jnp.where(col == i, cz, az)
        return (dists, cx, cy, cz, ax, ay, az)

    dists0 = jnp.full((f, n), 1e10, jnp.float32)
    _, _, _, _, ax, ay, az = jax.lax.fori_loop(
        1, npoint, body, (dists0, cx, cy, cz, ax, ay, az))
    ax_ref[...] = ax
    ay_ref[...] = ay
    az_ref[...] = az


def _fps(frames, npoint):
    f, n, _ = frames.shape
    outs = pl.pallas_call(
        functools.partial(_fps_body, npoint=npoint),
        out_shape=[jax.ShapeDtypeStruct((f, npoint), jnp.float32)] * 3,
    )(frames[..., 0], frames[..., 1], frames[..., 2])
    return jnp.stack(outs, axis=-1)


def _bn_relu(ys, g, b, relu=True):
    cnt = sum(y.shape[0] for y in ys)
    mean = sum(jnp.sum(y, axis=0, keepdims=True) for y in ys) / cnt
    var = sum(jnp.sum((y - mean) ** 2, axis=0, keepdims=True) for y in ys) / cnt
    scale = g / jnp.sqrt(var + EPS)
    out = [(y - mean) * scale + b for y in ys]
    if relu:
        out = [jnp.maximum(y, 0.0) for y in out]
    return out


def _ball_gather(A, XT, Xc, r2, npts):
    m = A.shape[0]
    n = npts
    ax = A[:, 0:1]
    ay = A[:, 1:2]
    az = A[:, 2:3]
    xr = XT[0:1, :]
    yr = XT[1:2, :]
    zr = XT[2:3, :]
    d2 = (ax - xr) ** 2 + (ay - yr) ** 2 + (az - zr) ** 2
    mask = d2 < r2
    mf = jnp.where(mask, 1.0, 0.0)
    r_i = jax.lax.broadcasted_iota(jnp.int32, (n, n), 0)
    c_i = jax.lax.broadcasted_iota(jnp.int32, (n, n), 1)
    tri = jnp.where(r_i < c_i, 1.0, 0.0)
    rank = jnp.dot(mf, tri, precision=HIGH)
    cnt = jnp.sum(mf, axis=1, keepdims=True)
    lane = jax.lax.broadcasted_iota(jnp.int32, (m, n), 1)
    sel0 = jnp.where(mask & (rank == 0.0), 1.0, 0.0)
    e0 = jnp.where(lane == 0, 1.0, 0.0)
    parts = []
    for r in range(K):
        sr = jnp.where(mask & (rank == float(r)), 1.0, 0.0)
        oh = jnp.where(cnt > r, sr, sel0)
        oh = jnp.where(cnt == 0.0, e0, oh)
        parts.append(jnp.dot(oh, Xc, precision=HIGH))
    return jnp.concatenate(parts, axis=0)


def _conv_chain(inds, gfs, w, relu_last=True):
    d1 = _bn_relu([jnp.dot(v, w['wd'], precision=HIGH) for v in inds],
                  w['gd'], w['bd'])
    f1 = _bn_relu([jnp.dot(v, w['wf'], precision=HIGH) for v in gfs],
                  w['gf'], w['bf'])
    h = [a * c for a, c in zip(d1, f1)]
    h = _bn_relu([jnp.dot(v, w['w1'], precision=HIGH) for v in h],
                 w['g1'], w['b1'])
    h = _bn_relu([jnp.dot(v, w['w2'], precision=HIGH) for v in h],
                 w['g2'], w['b2'])
    return h


def _kmax(z, m):
    acc = z[0:m, :]
    for r in range(1, K):
        acc = jnp.maximum(acc, z[r * m:(r + 1) * m, :])
    return acc


def _conv1_body(anch_ref, xt_ref, xc_ref,
                wd_ref, gd_ref, bd_ref, wf_ref, gf_ref, bf_ref,
                w1_ref, g1_ref, b1_ref, w2_ref, g2_ref, b2_ref,
                out_ref, *, r2):
    w = dict(wd=wd_ref[...], gd=gd_ref[...], bd=bd_ref[...],
             wf=wf_ref[...], gf=gf_ref[...], bf=bf_ref[...],
             w1=w1_ref[...], g1=g1_ref[...], b1=b1_ref[...],
             w2=w2_ref[...], g2=g2_ref[...], b2=b2_ref[...])
    inds, gfs = [], []
    for b in range(B):
        A = anch_ref[0, b]
        XT = xt_ref[0, b]
        Xc = xc_ref[0, b]
        gath = _ball_gather(A, XT, Xc, r2, N)
        anch_t = jnp.concatenate([A] * K, axis=0)
        disp = gath[:, 0:3] - anch_t
        tcol = jnp.zeros((K * M1, 1), jnp.float32)
        inds.append(jnp.concatenate([disp, tcol], axis=1))
        gfs.append(gath[:, 3:5])
    zs = _conv_chain(inds, gfs, w)
    for b in range(B):
        out_ref[0, b] = _kmax(zs[b], M1)


def _conv2_body(anch_ref, xt_ref, xc_ref,
                wd_ref, gd_ref, bd_ref, wf_ref, gf_ref, bf_ref,
                w1_ref, g1_ref, b1_ref, w2_ref, g2_ref, b2_ref,
                out_ref, *, r2):
    pr = pl.program_id(0)
    dtf = (pr % 3 - 1).astype(jnp.float32)
    w = dict(wd=wd_ref[...], gd=gd_ref[...], bd=bd_ref[...],
             wf=wf_ref[...], gf=gf_ref[...], bf=bf_ref[...],
             w1=w1_ref[...], g1=g1_ref[...], b1=b1_ref[...],
             w2=w2_ref[...], g2=g2_ref[...], b2=b2_ref[...])
    inds, gfs = [], []
    for b in range(B):
        A = anch_ref[0, b]
        XT = xt_ref[0, b]
        Xc = xc_ref[0, b]
        gath = _ball_gather(A, XT, Xc, r2, M1)
        anch_t = jnp.concatenate([A] * K, axis=0)
        disp = gath[:, 0:3] - anch_t
        tcol = jnp.full((K * M2, 1), 1.0, jnp.float32) * dtf
        inds.append(jnp.concatenate([disp, tcol], axis=1))
        gfs.append(gath[:, 3:])
    zs = _conv_chain(inds, gfs, w)
    first = pr % 3 == 0
    for b in range(B):
        res = _kmax(zs[b], M2)

        @pl.when(first)
        def _(b=b, res=res):
            out_ref[0, b] = res

        @pl.when(jnp.logical_not(first))
        def _(b=b, res=res):
            out_ref[0, b] = out_ref[0, b] + res


def _head_body(x2_ref, wp_ref, bias_ref, g_ref, b_ref, f2_ref, out_ref):
    X = x2_ref[...]
    pe = jnp.dot(X, wp_ref[...], precision=HIGH) + bias_ref[...]
    p = X.shape[0]
    mean = jnp.sum(pe, axis=0, keepdims=True) / p
    var = jnp.sum((pe - mean) ** 2, axis=0, keepdims=True) / p
    pe = (pe - mean) * (g_ref[...] / jnp.sqrt(var + EPS)) + b_ref[...]
    emb = pe + f2_ref[...]
    for g in range(T * B):
        blk = emb[g * M2:(g + 1) * M2, :]
        out_ref[g:g + 1, :] = jnp.max(blk, axis=0, keepdims=True)


def _cbr_weights(p):
    return (p['W'].T, p['g'].reshape(1, -1), p['b'].reshape(1, -1))


def _layer_weights(lp):
    wd, gd, bd = _cbr_weights(lp['conv_d'])
    wf, gf, bf = _cbr_weights(lp['conv_f'])
    w1, g1, b1 = _cbr_weights(lp['mlp'][0])
    w2, g2, b2 = _cbr_weights(lp['mlp'][1])
    return [wd, gd, bd, wf, gf, bf, w1, g1, b1, w2, g2, b2]


def kernel(xyzs, features, params):
    xyz_tb = jnp.transpose(xyzs, (1, 0, 2, 3))
    feat_tb = jnp.transpose(features, (1, 0, 3, 2))

    a1 = _fps(xyz_tb.reshape(T * B, N, 3), M1)
    A1 = a1.reshape(T, B, M1, 3)

    xt1 = jnp.transpose(xyz_tb, (0, 1, 3, 2))
    xc1 = jnp.concatenate([xyz_tb, feat_tb], axis=-1)

    wl1 = _layer_weights(params['conv1'])
    wspecs = [pl.BlockSpec(w.shape, lambda t: (0, 0)) for w in wl1]
    f1 = pl.pallas_call(
        functools.partial(_conv1_body, r2=float(RADIUS * RADIUS)),
        grid=(T,),
        in_specs=[
            pl.BlockSpec((1, B, M1, 3), lambda t: (t, 0, 0, 0)),
            pl.BlockSpec((1, B, 3, N), lambda t: (t, 0, 0, 0)),
            pl.BlockSpec((1, B, N, 5), lambda t: (t, 0, 0, 0)),
        ] + wspecs,
        out_specs=pl.BlockSpec((1, B, M1, 128), lambda t: (t, 0, 0, 0)),
        out_shape=jax.ShapeDtypeStruct((T, B, M1, 128), jnp.float32),
    )(A1, xt1, xc1, *wl1)

    a2 = _fps(A1.reshape(T * B, M1, 3), M2)
    A2 = a2.reshape(T, B, M2, 3)

    xt2 = jnp.transpose(A1, (0, 1, 3, 2))
    xc2 = jnp.concatenate([A1, f1], axis=-1)

    wl2 = _layer_weights(params['conv2'])
    wspecs2 = [pl.BlockSpec(w.shape, lambda p: (0, 0)) for w in wl2]
    f2 = pl.pallas_call(
        functools.partial(_conv2_body, r2=float(4.0 * RADIUS * RADIUS)),
        grid=(12,),
        in_specs=[
            pl.BlockSpec((1, B, M2, 3), lambda p: (p // 3, 0, 0, 0)),
            pl.BlockSpec((1, B, 3, M1),
                         lambda p: (jnp.clip(p // 3 + p % 3 - 1, 0, T - 1),
                                    0, 0, 0)),
            pl.BlockSpec((1, B, M1, 131),
                         lambda p: (jnp.clip(p // 3 + p % 3 - 1, 0, T - 1),
                                    0, 0, 0)),
        ] + wspecs2,
        out_specs=pl.BlockSpec((1, B, M2, EMBED), lambda p: (p // 3, 0, 0, 0)),
        out_shape=jax.ShapeDtypeStruct((T, B, M2, EMBED), jnp.float32),
        compiler_params=pltpu.CompilerParams(
            dimension_semantics=("arbitrary",)),
    )(A2, xt2, xc2, *wl2)

    X2 = A2.reshape(T * B * M2, 3)
    F2 = f2.reshape(T * B * M2, EMBED)
    pos = params['pos']
    out = pl.pallas_call(
        _head_body,
        out_shape=jax.ShapeDtypeStruct((T * B, EMBED), jnp.float32),
    )(X2, pos['W'].T, pos['bias'].reshape(1, -1),
      pos['g'].reshape(1, -1), pos['b'].reshape(1, -1), F2)

    return jnp.transpose(out.reshape(T, B, EMBED), (1, 0, 2))


# speedup vs baseline: 13.4863x; 1.6255x over previous
import functools

import jax
import jax.numpy as jnp
import numpy as np
from jax.experimental import pallas as pl
from jax.experimental.pallas import tpu as pltpu

RADIUS = 0.3
K = 16
B, T, N = 2, 4, 1024
M1, M2 = 128, 32
EMBED = 1024
HIGH = jax.lax.Precision.HIGHEST
EPS = 1e-5


def _fps_body(x_ref, y_ref, z_ref, ax_ref, ay_ref, az_ref, *, npoint):
    x = x_ref[...]
    y = y_ref[...]
    z = z_ref[...]
    f, n = x.shape
    col = jax.lax.broadcasted_iota(jnp.int32, (f, npoint), 1)
    lane = jax.lax.broadcasted_iota(jnp.int32, (f, n), 1)
    cx = x[:, 0:1]
    cy = y[:, 0:1]
    cz = z[:, 0:1]
    zero = jnp.zeros((f, npoint), jnp.float32)
    ax = jnp.where(col == 0, cx, zero)
    ay = jnp.where(col == 0, cy, zero)
    az = jnp.where(col == 0, cz, zero)

    def body(i, carry):
        dists, cx, cy, cz, ax, ay, az = carry
        d = (x - cx) ** 2 + (y - cy) ** 2 + (z - cz) ** 2
        dists = jnp.minimum(dists, d)
        m = jnp.max(dists, axis=1, keepdims=True)
        idx = jnp.min(jnp.where(dists == m, lane, n), axis=1, keepdims=True)
        oh = lane == idx
        cx = jnp.sum(jnp.where(oh, x, 0.0), axis=1, keepdims=True)
        cy = jnp.sum(jnp.where(oh, y, 0.0), axis=1, keepdims=True)
        cz = jnp.sum(jnp.where(oh, z, 0.0), axis=1, keepdims=True)
        ax = jnp.where(col == i, cx, ax)
        ay = jnp.where(col == i, cy, ay)
        az = jnp.where(col == i, cz, az)
        return (dists, cx, cy, cz, ax, ay, az)

    dists0 = jnp.full((f, n), 1e10, jnp.float32)
    _, _, _, _, ax, ay, az = jax.lax.fori_loop(
        1, npoint, body, (dists0, cx, cy, cz, ax, ay, az))
    ax_ref[...] = ax
    ay_ref[...] = ay
    az_ref[...] = az


def _fps(frames, npoint):
    f, n, _ = frames.shape
    outs = pl.pallas_call(
        functools.partial(_fps_body, npoint=npoint),
        out_shape=[jax.ShapeDtypeStruct((f, npoint), jnp.float32)] * 3,
    )(frames[..., 0], frames[..., 1], frames[..., 2])
    return jnp.stack(outs, axis=-1)


def _bn_relu(ys, g, b, relu=True):
    cnt = sum(y.shape[0] for y in ys)
    mean = sum(jnp.sum(y, axis=0, keepdims=True) for y in ys) / cnt
    var = sum(jnp.sum((y - mean) ** 2, axis=0, keepdims=True) for y in ys) / cnt
    scale = g / jnp.sqrt(var + EPS)
    out = [(y - mean) * scale + b for y in ys]
    if relu:
        out = [jnp.maximum(y, 0.0) for y in out]
    return out


def _ball_gather(A, XT, Xc, r2, npts):
    m = A.shape[0]
    n = npts
    ax = A[:, 0:1]
    ay = A[:, 1:2]
    az = A[:, 2:3]
    xr = XT[0:1, :]
    yr = XT[1:2, :]
    zr = XT[2:3, :]
    d2 = (ax - xr) ** 2 + (ay - yr) ** 2 + (az - zr) ** 2
    mask = d2 < r2
    mf = jnp.where(mask, 1.0, 0.0)
    r_i = jax.lax.broadcasted_iota(jnp.int32, (n, n), 0)
    c_i = jax.lax.broadcasted_iota(jnp.int32, (n, n), 1)
    tri = jnp.where(r_i < c_i, 1.0, 0.0)
    rank = jnp.dot(mf, tri)
    cnt = jnp.sum(mf, axis=1, keepdims=True)
    lane = jax.lax.broadcasted_iota(jnp.int32, (m, n), 1)
    sel0 = jnp.where(mask & (rank == 0.0), 1.0, 0.0)
    e0 = jnp.where(lane == 0, 1.0, 0.0)
    parts = []
    for r in range(K):
        sr = jnp.where(mask & (rank == float(r)), 1.0, 0.0)
        oh = jnp.where(cnt > r, sr, sel0)
        oh = jnp.where(cnt == 0.0, e0, oh)
        parts.append(oh)
    G = jnp.concatenate(parts, axis=0)
    return jnp.dot(G, Xc, precision=HIGH)


def _conv_chain(inds, gfs, w, relu_last=True):
    d1 = _bn_relu([jnp.dot(v, w['wd']) for v in inds], w['gd'], w['bd'])
    f1 = _bn_relu([jnp.dot(v, w['wf']) for v in gfs], w['gf'], w['bf'])
    h = [a * c for a, c in zip(d1, f1)]
    h = _bn_relu([jnp.dot(v, w['w1']) for v in h], w['g1'], w['b1'])
    h = _bn_relu([jnp.dot(v, w['w2']) for v in h], w['g2'], w['b2'])
    return h


def _kmax(z, m):
    acc = z[0:m, :]
    for r in range(1, K):
        acc = jnp.maximum(acc, z[r * m:(r + 1) * m, :])
    return acc


def _conv1_body(anch_ref, xt_ref, xc_ref,
                wd_ref, gd_ref, bd_ref, wf_ref, gf_ref, bf_ref,
                w1_ref, g1_ref, b1_ref, w2_ref, g2_ref, b2_ref,
                out_ref, *, r2):
    w = dict(wd=wd_ref[...], gd=gd_ref[...], bd=bd_ref[...],
             wf=wf_ref[...], gf=gf_ref[...], bf=bf_ref[...],
             w1=w1_ref[...], g1=g1_ref[...], b1=b1_ref[...],
             w2=w2_ref[...], g2=g2_ref[...], b2=b2_ref[...])
    inds, gfs = [], []
    for b in range(B):
        A = anch_ref[0, b]
        XT = xt_ref[0, b]
        Xc = xc_ref[0, b]
        gath = _ball_gather(A, XT, Xc, r2, N)
        anch_t = jnp.concatenate([A] * K, axis=0)
        disp = gath[:, 0:3] - anch_t
        tcol = jnp.zeros((K * M1, 1), jnp.float32)
        inds.append(jnp.concatenate([disp, tcol], axis=1))
        gfs.append(gath[:, 3:5])
    zs = _conv_chain(inds, gfs, w)
    for b in range(B):
        out_ref[0, b] = _kmax(zs[b], M1)


def _conv2_body(anch_ref, xt_ref, xc_ref,
                wd_ref, gd_ref, bd_ref, wf_ref, gf_ref, bf_ref,
                w1_ref, g1_ref, b1_ref, w2_ref, g2_ref, b2_ref,
                out_ref, *, r2):
    pr = pl.program_id(0)
    dtf = (pr % 3 - 1).astype(jnp.float32)
    w = dict(wd=wd_ref[...], gd=gd_ref[...], bd=bd_ref[...],
             wf=wf_ref[...], gf=gf_ref[...], bf=bf_ref[...],
             w1=w1_ref[...], g1=g1_ref[...], b1=b1_ref[...],
             w2=w2_ref[...], g2=g2_ref[...], b2=b2_ref[...])
    inds, gfs = [], []
    for b in range(B):
        A = anch_ref[0, b]
        XT = xt_ref[0, b]
        Xc = xc_ref[0, b]
        gath = _ball_gather(A, XT, Xc, r2, M1)
        anch_t = jnp.concatenate([A] * K, axis=0)
        disp = gath[:, 0:3] - anch_t
        tcol = jnp.full((K * M2, 1), 1.0, jnp.float32) * dtf
        inds.append(jnp.concatenate([disp, tcol], axis=1))
        gfs.append(gath[:, 3:])
    zs = _conv_chain(inds, gfs, w)
    first = pr % 3 == 0
    for b in range(B):
        res = _kmax(zs[b], M2)

        @pl.when(first)
        def _(b=b, res=res):
            out_ref[0, b] = res

        @pl.when(jnp.logical_not(first))
        def _(b=b, res=res):
            out_ref[0, b] = out_ref[0, b] + res


def _head_body(x2_ref, wp_ref, bias_ref, g_ref, b_ref, f2_ref, out_ref):
    X = x2_ref[...]
    pe = jnp.dot(X, wp_ref[...], precision=HIGH) + bias_ref[...]
    p = X.shape[0]
    mean = jnp.sum(pe, axis=0, keepdims=True) / p
    var = jnp.sum((pe - mean) ** 2, axis=0, keepdims=True) / p
    pe = (pe - mean) * (g_ref[...] / jnp.sqrt(var + EPS)) + b_ref[...]
    emb = pe + f2_ref[...]
    for g in range(T * B):
        blk = emb[g * M2:(g + 1) * M2, :]
        out_ref[g:g + 1, :] = jnp.max(blk, axis=0, keepdims=True)


def _cbr_weights(p):
    return (p['W'].T, p['g'].reshape(1, -1), p['b'].reshape(1, -1))


def _layer_weights(lp):
    wd, gd, bd = _cbr_weights(lp['conv_d'])
    wf, gf, bf = _cbr_weights(lp['conv_f'])
    w1, g1, b1 = _cbr_weights(lp['mlp'][0])
    w2, g2, b2 = _cbr_weights(lp['mlp'][1])
    return [wd, gd, bd, wf, gf, bf, w1, g1, b1, w2, g2, b2]


def kernel(xyzs, features, params):
    xyz_tb = jnp.transpose(xyzs, (1, 0, 2, 3))
    feat_tb = jnp.transpose(features, (1, 0, 3, 2))

    a1 = _fps(xyz_tb.reshape(T * B, N, 3), M1)
    A1 = a1.reshape(T, B, M1, 3)

    xt1 = jnp.transpose(xyz_tb, (0, 1, 3, 2))
    xc1 = jnp.concatenate([xyz_tb, feat_tb], axis=-1)

    wl1 = _layer_weights(params['conv1'])
    wspecs = [pl.BlockSpec(w.shape, lambda t: (0, 0)) for w in wl1]
    f1 = pl.pallas_call(
        functools.partial(_conv1_body, r2=float(RADIUS * RADIUS)),
        grid=(T,),
        in_specs=[
            pl.BlockSpec((1, B, M1, 3), lambda t: (t, 0, 0, 0)),
            pl.BlockSpec((1, B, 3, N), lambda t: (t, 0, 0, 0)),
            pl.BlockSpec((1, B, N, 5), lambda t: (t, 0, 0, 0)),
        ] + wspecs,
        out_specs=pl.BlockSpec((1, B, M1, 128), lambda t: (t, 0, 0, 0)),
        out_shape=jax.ShapeDtypeStruct((T, B, M1, 128), jnp.float32),
    )(A1, xt1, xc1, *wl1)

    a2 = _fps(A1.reshape(T * B, M1, 3), M2)
    A2 = a2.reshape(T, B, M2, 3)

    xt2 = jnp.transpose(A1, (0, 1, 3, 2))
    xc2 = jnp.concatenate([A1, f1], axis=-1)

    wl2 = _layer_weights(params['conv2'])
    wspecs2 = [pl.BlockSpec(w.shape, lambda p: (0, 0)) for w in wl2]
    f2 = pl.pallas_call(
        functools.partial(_conv2_body, r2=float(4.0 * RADIUS * RADIUS)),
        grid=(12,),
        in_specs=[
            pl.BlockSpec((1, B, M2, 3), lambda p: (p // 3, 0, 0, 0)),
            pl.BlockSpec((1, B, 3, M1),
                         lambda p: (jnp.clip(p // 3 + p % 3 - 1, 0, T - 1),
                                    0, 0, 0)),
            pl.BlockSpec((1, B, M1, 131),
                         lambda p: (jnp.clip(p // 3 + p % 3 - 1, 0, T - 1),
                                    0, 0, 0)),
        ] + wspecs2,
        out_specs=pl.BlockSpec((1, B, M2, EMBED), lambda p: (p // 3, 0, 0, 0)),
        out_shape=jax.ShapeDtypeStruct((T, B, M2, EMBED), jnp.float32),
        compiler_params=pltpu.CompilerParams(
            dimension_semantics=("arbitrary",)),
    )(A2, xt2, xc2, *wl2)

    X2 = A2.reshape(T * B * M2, 3)
    F2 = f2.reshape(T * B * M2, EMBED)
    pos = params['pos']
    out = pl.pallas_call(
        _head_body,
        out_shape=jax.ShapeDtypeStruct((T * B, EMBED), jnp.float32),
    )(X2, pos['W'].T, pos['bias'].reshape(1, -1),
      pos['g'].reshape(1, -1), pos['b'].reshape(1, -1), F2)

    return jnp.transpose(out.reshape(T, B, EMBED), (1, 0, 2))


# speedup vs baseline: 17.6514x; 1.3088x over previous
import functools

import jax
import jax.numpy as jnp
import numpy as np
from jax.experimental import pallas as pl
from jax.experimental.pallas import tpu as pltpu

RADIUS = 0.3
K = 16
B, T, N = 2, 4, 1024
M1, M2 = 128, 32
EMBED = 1024
HIGH = jax.lax.Precision.HIGHEST
EPS = 1e-5


def _fps_body(x_ref, y_ref, z_ref, ax_ref, ay_ref, az_ref, *, npoint):
    x = x_ref[...]
    y = y_ref[...]
    z = z_ref[...]
    f, n = x.shape
    col = jax.lax.broadcasted_iota(jnp.int32, (f, npoint), 1)
    lane = jax.lax.broadcasted_iota(jnp.int32, (f, n), 1)
    cx = x[:, 0:1]
    cy = y[:, 0:1]
    cz = z[:, 0:1]
    zero = jnp.zeros((f, npoint), jnp.float32)
    ax = jnp.where(col == 0, cx, zero)
    ay = jnp.where(col == 0, cy, zero)
    az = jnp.where(col == 0, cz, zero)

    def body(i, carry):
        dists, cx, cy, cz, ax, ay, az = carry
        d = (x - cx) ** 2 + (y - cy) ** 2 + (z - cz) ** 2
        dists = jnp.minimum(dists, d)
        m = jnp.max(dists, axis=1, keepdims=True)
        idx = jnp.min(jnp.where(dists == m, lane, n), axis=1, keepdims=True)
        oh = lane == idx
        cx = jnp.sum(jnp.where(oh, x, 0.0), axis=1, keepdims=True)
        cy = jnp.sum(jnp.where(oh, y, 0.0), axis=1, keepdims=True)
        cz = jnp.sum(jnp.where(oh, z, 0.0), axis=1, keepdims=True)
        ax = jnp.where(col == i, cx, ax)
        ay = jnp.where(col == i, cy, ay)
        az = jnp.where(col == i, cz, az)
        return (dists, cx, cy, cz, ax, ay, az)

    dists0 = jnp.full((f, n), 1e10, jnp.float32)
    _, _, _, _, ax, ay, az = jax.lax.fori_loop(
        1, npoint, body, (dists0, cx, cy, cz, ax, ay, az))
    ax_ref[...] = ax
    ay_ref[...] = ay
    az_ref[...] = az


def _fps(frames, npoint):
    f, n, _ = frames.shape
    outs = pl.pallas_call(
        functools.partial(_fps_body, npoint=npoint),
        out_shape=[jax.ShapeDtypeStruct((f, npoint), jnp.float32)] * 3,
    )(frames[..., 0], frames[..., 1], frames[..., 2])
    return jnp.stack(outs, axis=-1)


def _bn_relu(ys, g, b, relu=True):
    cnt = sum(y.shape[0] for y in ys)
    mean = sum(jnp.sum(y, axis=0, keepdims=True) for y in ys) / cnt
    var = sum(jnp.sum((y - mean) ** 2, axis=0, keepdims=True) for y in ys) / cnt
    scale = g / jnp.sqrt(var + EPS)
    out = [(y - mean) * scale + b for y in ys]
    if relu:
        out = [jnp.maximum(y, 0.0) for y in out]
    return out


def _ball_gather(A, XT, Xc, r2, npts):
    m = A.shape[0]
    n = npts
    ax = A[:, 0:1]
    ay = A[:, 1:2]
    az = A[:, 2:3]
    xr = XT[0:1, :]
    yr = XT[1:2, :]
    zr = XT[2:3, :]
    d2 = (ax - xr) ** 2 + (ay - yr) ** 2 + (az - zr) ** 2
    mask = d2 < r2
    mf = jnp.where(mask, 1.0, 0.0)
    r_i = jax.lax.broadcasted_iota(jnp.int32, (n, n), 0)
    c_i = jax.lax.broadcasted_iota(jnp.int32, (n, n), 1)
    tri = jnp.where(r_i < c_i, 1.0, 0.0)
    rank = jnp.dot(mf, tri)
    cnt = jnp.sum(mf, axis=1, keepdims=True)
    lane = jax.lax.broadcasted_iota(jnp.int32, (m, n), 1)
    sel0 = jnp.where(mask & (rank == 0.0), 1.0, 0.0)
    e0 = jnp.where(lane == 0, 1.0, 0.0)
    parts = []
    for r in range(K):
        sr = jnp.where(mask & (rank == float(r)), 1.0, 0.0)
        oh = jnp.where(cnt > r, sr, sel0)
        oh = jnp.where(cnt == 0.0, e0, oh)
        parts.append(oh)
    G = jnp.concatenate(parts, axis=0)
    return jnp.dot(G, Xc)


def _conv_chain(inds, gfs, w, relu_last=True):
    d1 = _bn_relu([jnp.dot(v, w['wd']) for v in inds], w['gd'], w['bd'])
    f1 = _bn_relu([jnp.dot(v, w['wf']) for v in gfs], w['gf'], w['bf'])
    h = [a * c for a, c in zip(d1, f1)]
    h = _bn_relu([jnp.dot(v, w['w1']) for v in h], w['g1'], w['b1'])
    h = _bn_relu([jnp.dot(v, w['w2']) for v in h], w['g2'], w['b2'])
    return h


def _kmax(z, m):
    acc = z[0:m, :]
    for r in range(1, K):
        acc = jnp.maximum(acc, z[r * m:(r + 1) * m, :])
    return acc


def _conv1_body(anch_ref, xt_ref, xc_ref,
                wd_ref, gd_ref, bd_ref, wf_ref, gf_ref, bf_ref,
                w1_ref, g1_ref, b1_ref, w2_ref, g2_ref, b2_ref,
                out_ref, *, r2):
    w = dict(wd=wd_ref[...], gd=gd_ref[...], bd=bd_ref[...],
             wf=wf_ref[...], gf=gf_ref[...], bf=bf_ref[...],
             w1=w1_ref[...], g1=g1_ref[...], b1=b1_ref[...],
             w2=w2_ref[...], g2=g2_ref[...], b2=b2_ref[...])
    inds, gfs = [], []
    for b in range(B):
        A = anch_ref[0, b]
        XT = xt_ref[0, b]
        Xc = xc_ref[0, b]
        gath = _ball_gather(A, XT, Xc, r2, N)
        anch_t = jnp.concatenate([A] * K, axis=0)
        disp = gath[:, 0:3] - anch_t
        tcol = jnp.zeros((K * M1, 1), jnp.float32)
        inds.append(jnp.concatenate([disp, tcol], axis=1))
        gfs.append(gath[:, 3:5])
    zs = _conv_chain(inds, gfs, w)
    for b in range(B):
        out_ref[0, b] = _kmax(zs[b], M1)


def _conv2_body(anch_ref, xt_ref, xc_ref,
                wd_ref, gd_ref, bd_ref, wf_ref, gf_ref, bf_ref,
                w1_ref, g1_ref, b1_ref, w2_ref, g2_ref, b2_ref,
                out_ref, *, r2):
    pr = pl.program_id(0)
    dtf = (pr % 3 - 1).astype(jnp.float32)
    w = dict(wd=wd_ref[...], gd=gd_ref[...], bd=bd_ref[...],
             wf=wf_ref[...], gf=gf_ref[...], bf=bf_ref[...],
             w1=w1_ref[...], g1=g1_ref[...], b1=b1_ref[...],
             w2=w2_ref[...], g2=g2_ref[...], b2=b2_ref[...])
    inds, gfs = [], []
    for b in range(B):
        A = anch_ref[0, b]
        XT = xt_ref[0, b]
        Xc = xc_ref[0, b]
        gath = _ball_gather(A, XT, Xc, r2, M1)
        anch_t = jnp.concatenate([A] * K, axis=0)
        disp = gath[:, 0:3] - anch_t
        tcol = jnp.full((K * M2, 1), 1.0, jnp.float32) * dtf
        inds.append(jnp.concatenate([disp, tcol], axis=1))
        gfs.append(gath[:, 3:])
    zs = _conv_chain(inds, gfs, w)
    first = pr % 3 == 0
    for b in range(B):
        res = _kmax(zs[b], M2)

        @pl.when(first)
        def _(b=b, res=res):
            out_ref[0, b] = res

        @pl.when(jnp.logical_not(first))
        def _(b=b, res=res):
            out_ref[0, b] = out_ref[0, b] + res


def _head_body(x2_ref, wp_ref, bias_ref, g_ref, b_ref, f2_ref, out_ref):
    X = x2_ref[...]
    pe = jnp.dot(X, wp_ref[...]) + bias_ref[...]
    p = X.shape[0]
    mean = jnp.sum(pe, axis=0, keepdims=True) / p
    var = jnp.sum((pe - mean) ** 2, axis=0, keepdims=True) / p
    pe = (pe - mean) * (g_ref[...] / jnp.sqrt(var + EPS)) + b_ref[...]
    emb = pe + f2_ref[...]
    for g in range(T * B):
        blk = emb[g * M2:(g + 1) * M2, :]
        out_ref[g:g + 1, :] = jnp.max(blk, axis=0, keepdims=True)


def _cbr_weights(p):
    return (p['W'].T, p['g'].reshape(1, -1), p['b'].reshape(1, -1))


def _layer_weights(lp):
    wd, gd, bd = _cbr_weights(lp['conv_d'])
    wf, gf, bf = _cbr_weights(lp['conv_f'])
    w1, g1, b1 = _cbr_weights(lp['mlp'][0])
    w2, g2, b2 = _cbr_weights(lp['mlp'][1])
    return [wd, gd, bd, wf, gf, bf, w1, g1, b1, w2, g2, b2]


def kernel(xyzs, features, params):
    xyz_tb = jnp.transpose(xyzs, (1, 0, 2, 3))
    feat_tb = jnp.transpose(features, (1, 0, 3, 2))

    a1 = _fps(xyz_tb.reshape(T * B, N, 3), M1)
    A1 = a1.reshape(T, B, M1, 3)

    xt1 = jnp.transpose(xyz_tb, (0, 1, 3, 2))
    xc1 = jnp.concatenate([xyz_tb, feat_tb], axis=-1)

    wl1 = _layer_weights(params['conv1'])
    wspecs = [pl.BlockSpec(w.shape, lambda t: (0, 0)) for w in wl1]
    f1 = pl.pallas_call(
        functools.partial(_conv1_body, r2=float(RADIUS * RADIUS)),
        grid=(T,),
        in_specs=[
            pl.BlockSpec((1, B, M1, 3), lambda t: (t, 0, 0, 0)),
            pl.BlockSpec((1, B, 3, N), lambda t: (t, 0, 0, 0)),
            pl.BlockSpec((1, B, N, 5), lambda t: (t, 0, 0, 0)),
        ] + wspecs,
        out_specs=pl.BlockSpec((1, B, M1, 128), lambda t: (t, 0, 0, 0)),
        out_shape=jax.ShapeDtypeStruct((T, B, M1, 128), jnp.float32),
    )(A1, xt1, xc1, *wl1)

    a2 = _fps(A1.reshape(T * B, M1, 3), M2)
    A2 = a2.reshape(T, B, M2, 3)

    xt2 = jnp.transpose(A1, (0, 1, 3, 2))
    xc2 = jnp.concatenate([A1, f1], axis=-1)

    wl2 = _layer_weights(params['conv2'])
    wspecs2 = [pl.BlockSpec(w.shape, lambda p: (0, 0)) for w in wl2]
    f2 = pl.pallas_call(
        functools.partial(_conv2_body, r2=float(4.0 * RADIUS * RADIUS)),
        grid=(12,),
        in_specs=[
            pl.BlockSpec((1, B, M2, 3), lambda p: (p // 3, 0, 0, 0)),
            pl.BlockSpec((1, B, 3, M1),
                         lambda p: (jnp.clip(p // 3 + p % 3 - 1, 0, T - 1),
                                    0, 0, 0)),
            pl.BlockSpec((1, B, M1, 131),
                         lambda p: (jnp.clip(p // 3 + p % 3 - 1, 0, T - 1),
                                    0, 0, 0)),
        ] + wspecs2,
        out_specs=pl.BlockSpec((1, B, M2, EMBED), lambda p: (p // 3, 0, 0, 0)),
        out_shape=jax.ShapeDtypeStruct((T, B, M2, EMBED), jnp.float32),
        compiler_params=pltpu.CompilerParams(
            dimension_semantics=("arbitrary",)),
    )(A2, xt2, xc2, *wl2)

    X2 = A2.reshape(T * B * M2, 3)
    F2 = f2.reshape(T * B * M2, EMBED)
    pos = params['pos']
    out = pl.pallas_call(
        _head_body,
        out_shape=jax.ShapeDtypeStruct((T * B, EMBED), jnp.float32),
    )(X2, pos['W'].T, pos['bias'].reshape(1, -1),
      pos['g'].reshape(1, -1), pos['b'].reshape(1, -1), F2)

    return jnp.transpose(out.reshape(T, B, EMBED), (1, 0, 2))
